# Initial kernel scaffold; baseline (speedup 1.0000x reference)
import jax
import jax.numpy as jnp
from jax.experimental import pallas as pl


def kernel(x, block, W_l1, b_l1, W_r1, b_r1, W_l2, b_l2, W_r2, b_r2):
    raise NotImplementedError("write your pallas kernel here")


# speedup vs baseline: 1.1095x; 1.1095x over previous
import functools

import jax
import jax.numpy as jnp
from jax.experimental import pallas as pl

N = 10000
F = 128
BM = 400


def _layer_body(adj_ref, src_ref, srcr_ref, wlt_ref, wrt_ref, bias_ref,
                out_ref, *, final):
    acc = jnp.dot(adj_ref[...].astype(jnp.bfloat16),
                  src_ref[...].astype(jnp.bfloat16),
                  preferred_element_type=jnp.float32)
    r = jnp.dot(acc.astype(jnp.bfloat16), wlt_ref[...].astype(jnp.bfloat16),
                preferred_element_type=jnp.float32)
    r += jnp.dot(srcr_ref[...].astype(jnp.bfloat16),
                 wrt_ref[...].astype(jnp.bfloat16),
                 preferred_element_type=jnp.float32)
    r += bias_ref[...]
    if not final:
        denom = jnp.maximum(jnp.sum(jnp.abs(r), axis=1, keepdims=True), 1e-12)
        r = jnp.maximum(r / denom, 0.0)
    else:
        r = r - jnp.max(r, axis=1, keepdims=True)
        r = r - jnp.log(jnp.sum(jnp.exp(r), axis=1, keepdims=True))
    out_ref[...] = r


def _layer(adj, src, wlt, wrt, bias, final):
    return pl.pallas_call(
        functools.partial(_layer_body, final=final),
        grid=(N // BM,),
        in_specs=[
            pl.BlockSpec((BM, N), lambda i: (i, 0)),
            pl.BlockSpec((N, F), lambda i: (0, 0)),
            pl.BlockSpec((BM, F), lambda i: (i, 0)),
            pl.BlockSpec((F, F), lambda i: (0, 0)),
            pl.BlockSpec((F, F), lambda i: (0, 0)),
            pl.BlockSpec((1, F), lambda i: (0, 0)),
        ],
        out_specs=pl.BlockSpec((BM, F), lambda i: (i, 0)),
        out_shape=jax.ShapeDtypeStruct((N, F), jnp.float32),
    )(adj, src, src, wlt, wrt, bias)


def kernel(x, block, W_l1, b_l1, W_r1, b_r1, W_l2, b_l2, W_r2, b_r2):
    adj = block[0]
    b1 = (b_l1 + b_r1).reshape(1, F)
    b2 = (b_l2 + b_r2).reshape(1, F)
    h = _layer(adj, x, W_l1.T, W_r1.T, b1, final=False)
    return _layer(adj, h, W_l2.T, W_r2.T, b2, final=True)


# speedup vs baseline: 1.3271x; 1.1961x over previous
import jax
import jax.numpy as jnp
from jax.experimental import pallas as pl

N = 10000
F = 128
BM = 400
HSCALE = 64.0


def _layer1_body(adj_ref, src_ref, srcr_ref, wlt_ref, wrt_ref, bias_ref,
                 h_ref, h8_ref, adj8_ref):
    a = adj_ref[...]
    adj8_ref[...] = a.astype(jnp.float8_e4m3fn)
    acc = jnp.dot(a.astype(jnp.bfloat16), src_ref[...].astype(jnp.bfloat16),
                  preferred_element_type=jnp.float32)
    r = jnp.dot(acc.astype(jnp.bfloat16), wlt_ref[...].astype(jnp.bfloat16),
                preferred_element_type=jnp.float32)
    r += jnp.dot(srcr_ref[...].astype(jnp.bfloat16),
                 wrt_ref[...].astype(jnp.bfloat16),
                 preferred_element_type=jnp.float32)
    r += bias_ref[...]
    denom = jnp.maximum(jnp.sum(jnp.abs(r), axis=1, keepdims=True), 1e-12)
    r = jnp.maximum(r / denom, 0.0)
    h_ref[...] = r
    h8_ref[...] = (r * HSCALE).astype(jnp.float8_e4m3fn)


def _layer2_body(adj8_ref, h8_ref, srcr_ref, wlt_ref, wrt_ref, bias_ref,
                 out_ref):
    acc = jnp.dot(adj8_ref[...], h8_ref[...],
                  preferred_element_type=jnp.float32)
    r = jnp.dot(acc.astype(jnp.bfloat16), wlt_ref[...].astype(jnp.bfloat16),
                preferred_element_type=jnp.float32)
    r += jnp.dot(srcr_ref[...].astype(jnp.bfloat16),
                 wrt_ref[...].astype(jnp.bfloat16),
                 preferred_element_type=jnp.float32)
    r += bias_ref[...]
    r = r - jnp.max(r, axis=1, keepdims=True)
    r = r - jnp.log(jnp.sum(jnp.exp(r), axis=1, keepdims=True))
    out_ref[...] = r


_SMALL_SPECS = [
    pl.BlockSpec((BM, F), lambda i: (i, 0)),
    pl.BlockSpec((F, F), lambda i: (0, 0)),
    pl.BlockSpec((F, F), lambda i: (0, 0)),
    pl.BlockSpec((1, F), lambda i: (0, 0)),
]


def kernel(x, block, W_l1, b_l1, W_r1, b_r1, W_l2, b_l2, W_r2, b_r2):
    adj = block[0]
    b1 = (b_l1 + b_r1).reshape(1, F)
    b2 = (b_l2 + b_r2).reshape(1, F)

    h, h8, adj8 = pl.pallas_call(
        _layer1_body,
        grid=(N // BM,),
        in_specs=[
            pl.BlockSpec((BM, N), lambda i: (i, 0)),
            pl.BlockSpec((N, F), lambda i: (0, 0)),
        ] + _SMALL_SPECS,
        out_specs=[
            pl.BlockSpec((BM, F), lambda i: (i, 0)),
            pl.BlockSpec((BM, F), lambda i: (i, 0)),
            pl.BlockSpec((BM, N), lambda i: (i, 0)),
        ],
        out_shape=[
            jax.ShapeDtypeStruct((N, F), jnp.float32),
            jax.ShapeDtypeStruct((N, F), jnp.float8_e4m3fn),
            jax.ShapeDtypeStruct((N, N), jnp.float8_e4m3fn),
        ],
    )(adj, x, x, W_l1.T, W_r1.T, b1)

    return pl.pallas_call(
        _layer2_body,
        grid=(N // BM,),
        in_specs=[
            pl.BlockSpec((BM, N), lambda i: (i, 0)),
            pl.BlockSpec((N, F), lambda i: (0, 0)),
        ] + _SMALL_SPECS,
        out_specs=pl.BlockSpec((BM, F), lambda i: (i, 0)),
        out_shape=jax.ShapeDtypeStruct((N, F), jnp.float32),
    )(adj8, h8, h, W_l2.T / HSCALE, W_r2.T, b2)


# speedup vs baseline: 1.3491x; 1.0165x over previous
import jax
import jax.numpy as jnp
from jax.experimental import pallas as pl

N = 10000
F = 128
BM = 400
HSCALE = 64.0


def _layer1_body(adj_ref, src_ref, srcr_ref, wlt_ref, wrt_ref, bias_ref,
                 h8_ref, adj8_ref):
    a = adj_ref[...]
    adj8_ref[...] = a.astype(jnp.float8_e4m3fn)
    acc = jnp.dot(a.astype(jnp.bfloat16), src_ref[...].astype(jnp.bfloat16),
                  preferred_element_type=jnp.float32)
    r = jnp.dot(acc.astype(jnp.bfloat16), wlt_ref[...].astype(jnp.bfloat16),
                preferred_element_type=jnp.float32)
    r += jnp.dot(srcr_ref[...].astype(jnp.bfloat16),
                 wrt_ref[...].astype(jnp.bfloat16),
                 preferred_element_type=jnp.float32)
    r += bias_ref[...]
    denom = jnp.maximum(jnp.sum(jnp.abs(r), axis=1, keepdims=True), 1e-12)
    r = jnp.maximum(r / denom, 0.0)
    h8_ref[...] = (r * HSCALE).astype(jnp.float8_e4m3fn)


def _layer2_body(adj8_ref, h8_ref, srcr_ref, wlt_ref, wrt_ref, bias_ref,
                 out_ref):
    acc = jnp.dot(adj8_ref[...], h8_ref[...],
                  preferred_element_type=jnp.float32)
    r = jnp.dot(acc.astype(jnp.bfloat16), wlt_ref[...].astype(jnp.bfloat16),
                preferred_element_type=jnp.float32)
    r += jnp.dot(srcr_ref[...].astype(jnp.bfloat16),
                 wrt_ref[...].astype(jnp.bfloat16),
                 preferred_element_type=jnp.float32)
    r += bias_ref[...]
    r = r - jnp.max(r, axis=1, keepdims=True)
    r = r - jnp.log(jnp.sum(jnp.exp(r), axis=1, keepdims=True))
    out_ref[...] = r


_SMALL_SPECS = [
    pl.BlockSpec((BM, F), lambda i: (i, 0)),
    pl.BlockSpec((F, F), lambda i: (0, 0)),
    pl.BlockSpec((F, F), lambda i: (0, 0)),
    pl.BlockSpec((1, F), lambda i: (0, 0)),
]


def kernel(x, block, W_l1, b_l1, W_r1, b_r1, W_l2, b_l2, W_r2, b_r2):
    adj = block[0]
    b1 = (b_l1 + b_r1).reshape(1, F)
    b2 = (b_l2 + b_r2).reshape(1, F)

    h8, adj8 = pl.pallas_call(
        _layer1_body,
        grid=(N // BM,),
        in_specs=[
            pl.BlockSpec((BM, N), lambda i: (i, 0)),
            pl.BlockSpec((N, F), lambda i: (0, 0)),
        ] + _SMALL_SPECS,
        out_specs=[
            pl.BlockSpec((BM, F), lambda i: (i, 0)),
            pl.BlockSpec((BM, N), lambda i: (i, 0)),
        ],
        out_shape=[
            jax.ShapeDtypeStruct((N, F), jnp.float8_e4m3fn),
            jax.ShapeDtypeStruct((N, N), jnp.float8_e4m3fn),
        ],
    )(adj, x, x, W_l1.T, W_r1.T, b1)

    return pl.pallas_call(
        _layer2_body,
        grid=(N // BM,),
        in_specs=[
            pl.BlockSpec((BM, N), lambda i: (i, 0)),
            pl.BlockSpec((N, F), lambda i: (0, 0)),
        ] + _SMALL_SPECS,
        out_specs=pl.BlockSpec((BM, F), lambda i: (i, 0)),
        out_shape=jax.ShapeDtypeStruct((N, F), jnp.float32),
    )(adj8, h8, h8, W_l2.T / HSCALE, W_r2.T / HSCALE, b2)
